# baseline (speedup 1.0000x reference)
import functools

import jax
import jax.numpy as jnp
from jax import lax
from jax.experimental import pallas as pl
from jax.experimental.pallas import tpu as pltpu
from jax.experimental.pallas import tpu_sc as plsc

B = 256
MARGIN = 0.2
EPS = 1e-8
BIG = 1e30

NC = 2
NS = 16
NW = NC * NS
L = 16
NCHUNK = B // L

K_SC = 64
APW = K_SC // NW
PBUF = APW * (B - 1) + 2 * L
TCB = 8
TC_STEPS = (B - K_SC) // TCB


def _dist_kernel(x_ref, d_ref):
    x = x_ref[:, :]
    g = lax.dot_general(x, x, (((1,), (1,)), ((), ())),
                        preferred_element_type=jnp.float32)
    r = jnp.sum(x * x, axis=1)
    sq = r[:, None] + r[None, :] - 2.0 * g
    d_ref[:, :] = jnp.sqrt(jnp.maximum(sq, 1e-12))


_compute_dists = pl.pallas_call(
    _dist_kernel,
    out_shape=jax.ShapeDtypeStruct((B, B), jnp.float32),
)


@functools.partial(
    pl.kernel,
    out_type=jax.ShapeDtypeStruct((NW * L,), jnp.float32),
    mesh=plsc.VectorSubcoreMesh(core_axis_name="c", subcore_axis_name="s"),
    scratch_types=[
        pltpu.VMEM((APW, B), jnp.float32),
        pltpu.VMEM((B,), jnp.int32),
        pltpu.VMEM((APW * B,), jnp.float32),
        pltpu.VMEM((PBUF,), jnp.float32),
        pltpu.VMEM((PBUF,), jnp.int32),
        pltpu.VMEM((L,), jnp.float32),
    ],
    compiler_params=pltpu.CompilerParams(needs_layout_passes=False),
)
def _triplet_sc(d_hbm, y_hbm, out_hbm, d_v, y_v, nbuf, gdp, goff, stage):
    wid = lax.axis_index("s") * NC + lax.axis_index("c")
    base = wid * APW
    pltpu.sync_copy(y_hbm, y_v)
    pltpu.sync_copy(d_hbm.at[pl.ds(base, APW)], d_v)

    lane_iota = lax.iota(jnp.int32, L)
    zero_i = jnp.zeros((L,), jnp.int32)
    hinge_c = jnp.float32(MARGIN - EPS)

    base_splat = zero_i + base
    ya = [plsc.load_gather(y_v, [base_splat + i]) for i in range(APW)]

    pbases = [zero_i] * APW
    gbase = zero_i
    for j in range(NCHUNK):
        yj = y_v[pl.ds(j * L, L)]
        idxj = lane_iota + (j * L)
        for i in range(APW):
            dj = d_v[i, pl.ds(j * L, L)]
            same = yj == ya[i]
            posm = same & (idxj != base_splat + i)
            nbuf[pl.ds(i * B + j * L, L)] = jnp.where(same, jnp.float32(BIG), dj)
            dest = gbase + plsc.cumsum(posm.astype(jnp.int32)) - 1
            dest = jnp.where(posm, dest, 0)
            plsc.store_scatter(gdp, [dest], dj, mask=posm)
            plsc.store_scatter(goff, [dest], zero_i + (i * B), mask=posm)
            pc = plsc.all_reduce_population_count(posm)
            pbases[i] = pbases[i] + pc
            gbase = gbase + pc

    pairs = zero_i
    for i in range(APW):
        pairs = pairs + pbases[i] * (255 - pbases[i])

    tot = jnp.max(gbase)
    plsc.store_scatter(gdp, [zero_i + tot + lane_iota], jnp.full((L,), -BIG, jnp.float32))
    plsc.store_scatter(goff, [zero_i + tot + lane_iota], zero_i)

    def p_body(t, accs):
        a0, a1, a2, a3 = accs
        k0 = zero_i + 2 * t
        dp0 = plsc.load_gather(gdp, [k0])
        off0 = plsc.load_gather(goff, [k0])
        dp1 = plsc.load_gather(gdp, [k0 + 1])
        off1 = plsc.load_gather(goff, [k0 + 1])
        for j in range(NCHUNK):
            cidx = lane_iota + (j * L)
            nb0 = plsc.load_gather(nbuf, [off0 + cidx])
            nb1 = plsc.load_gather(nbuf, [off1 + cidx])
            h0 = jnp.maximum(dp0 - nb0 + hinge_c, 0.0)
            h1 = jnp.maximum(dp1 - nb1 + hinge_c, 0.0)
            if j % 2 == 0:
                a0 = a0 + h0
                a2 = a2 + h1
            else:
                a1 = a1 + h0
                a3 = a3 + h1
        return a0, a1, a2, a3

    zero_f = jnp.zeros((L,), jnp.float32)
    accs = lax.fori_loop(0, (tot + 1) // 2, p_body,
                         (zero_f, zero_f, zero_f, zero_f))
    acc = (accs[0] + accs[1]) + (accs[2] + accs[3])
    acc = acc + jnp.float32(EPS / L) * pairs.astype(jnp.float32)
    stage[...] = acc
    pltpu.sync_copy(stage, out_hbm.at[pl.ds(wid * L, L)])


def _tc_share_kernel(d_ref, ycol_ref, yrow_ref, ysmem_ref, out_ref):
    step = pl.program_id(0)
    dblk = d_ref[:, :]
    dt = dblk.T
    ycol = ycol_ref[:, :]
    yrow = yrow_ref[:, :]
    riota = lax.broadcasted_iota(jnp.int32, (B, 1), 0)
    hinge_c = jnp.float32(MARGIN - EPS)
    s = jnp.float32(0.0)
    for i in range(TCB):
        a = K_SC + step * TCB + i
        ya = ysmem_ref[a]
        posm = (ycol == ya) & (riota != a)
        pcol = jnp.where(posm, dt[:, i:i + 1] + hinge_c, jnp.float32(-BIG))
        nrow = jnp.where(yrow != ya, dblk[i:i + 1, :], jnp.float32(BIG))
        s = s + jnp.sum(jnp.maximum(pcol - nrow, 0.0))
        npos = jnp.sum(posm.astype(jnp.float32))
        s = s + jnp.float32(EPS) * npos * (255.0 - npos)
    prev = jnp.where(step == 0, 0.0, out_ref[0, 0])
    out_ref[0, 0] = prev + s


_tc_share = pl.pallas_call(
    _tc_share_kernel,
    grid=(TC_STEPS,),
    in_specs=[
        pl.BlockSpec((TCB, B), lambda s: (K_SC // TCB + s, 0)),
        pl.BlockSpec((B, 1), lambda s: (0, 0)),
        pl.BlockSpec((1, B), lambda s: (0, 0)),
        pl.BlockSpec(memory_space=pltpu.SMEM),
    ],
    out_specs=pl.BlockSpec(memory_space=pltpu.SMEM),
    out_shape=jax.ShapeDtypeStruct((1, 1), jnp.float32),
)


def kernel(x, y):
    d = _compute_dists(x)
    sc_partials = _triplet_sc(d, y)
    tc_partial = _tc_share(d, y.reshape(B, 1), y.reshape(1, B), y)
    return jnp.sum(sc_partials) + tc_partial[0, 0]


# speedup vs baseline: 1.4904x; 1.4904x over previous
import functools

import jax
import jax.numpy as jnp
from jax import lax
from jax.experimental import pallas as pl
from jax.experimental.pallas import tpu as pltpu
from jax.experimental.pallas import tpu_sc as plsc

B = 256
MARGIN = 0.2
EPS = 1e-8
BIG = 1e30

NC = 2
NS = 16
NW = NC * NS
L = 16
NCHUNK = B // L

K_SC = 64
APW = K_SC // NW
PBUF = APW * (B - 1) + 2 * L
TCB = 8
TC_STEPS = (B - K_SC) // TCB


def _dist_kernel(x_ref, d_ref):
    x = x_ref[:, :]
    g = lax.dot_general(x, x, (((1,), (1,)), ((), ())),
                        preferred_element_type=jnp.float32)
    r = jnp.sum(x * x, axis=1)
    sq = r[:, None] + r[None, :] - 2.0 * g
    d_ref[:, :] = jnp.sqrt(jnp.maximum(sq, 1e-12))


_compute_dists = pl.pallas_call(
    _dist_kernel,
    out_shape=jax.ShapeDtypeStruct((B, B), jnp.float32),
)


@functools.partial(
    pl.kernel,
    out_type=jax.ShapeDtypeStruct((NW * L,), jnp.float32),
    mesh=plsc.VectorSubcoreMesh(core_axis_name="c", subcore_axis_name="s"),
    scratch_types=[
        pltpu.VMEM((APW, B), jnp.float32),
        pltpu.VMEM((B,), jnp.int32),
        pltpu.VMEM((APW * B,), jnp.float32),
        pltpu.VMEM((PBUF,), jnp.float32),
        pltpu.VMEM((PBUF,), jnp.int32),
        pltpu.VMEM((L,), jnp.float32),
    ],
    compiler_params=pltpu.CompilerParams(needs_layout_passes=False),
)
def _triplet_sc(d_hbm, y_hbm, out_hbm, d_v, y_v, nbuf, gdp, goff, stage):
    wid = lax.axis_index("s") * NC + lax.axis_index("c")
    base = wid * APW
    pltpu.sync_copy(y_hbm, y_v)
    pltpu.sync_copy(d_hbm.at[pl.ds(base, APW)], d_v)

    lane_iota = lax.iota(jnp.int32, L)
    zero_i = jnp.zeros((L,), jnp.int32)
    hinge_c = jnp.float32(MARGIN - EPS)

    base_splat = zero_i + base
    ya = [plsc.load_gather(y_v, [base_splat + i]) for i in range(APW)]

    pbases = [zero_i] * APW
    gbase = zero_i
    for j in range(NCHUNK):
        yj = y_v[pl.ds(j * L, L)]
        idxj = lane_iota + (j * L)
        for i in range(APW):
            dj = d_v[i, pl.ds(j * L, L)]
            same = yj == ya[i]
            posm = same & (idxj != base_splat + i)
            nbuf[pl.ds(i * B + j * L, L)] = jnp.where(same, jnp.float32(BIG), dj)
            dest = gbase + plsc.cumsum(posm.astype(jnp.int32)) - 1
            dest = jnp.where(posm, dest, 0)
            plsc.store_scatter(gdp, [dest], dj, mask=posm)
            plsc.store_scatter(goff, [dest], zero_i + (i * B), mask=posm)
            pc = plsc.all_reduce_population_count(posm)
            pbases[i] = pbases[i] + pc
            gbase = gbase + pc

    pairs = zero_i
    for i in range(APW):
        pairs = pairs + pbases[i] * (255 - pbases[i])

    tot = jnp.max(gbase)
    plsc.store_scatter(gdp, [zero_i + tot + lane_iota], jnp.full((L,), -BIG, jnp.float32))
    plsc.store_scatter(goff, [zero_i + tot + lane_iota], zero_i)

    def p_body(t, accs):
        a0, a1, a2, a3 = accs
        k0 = zero_i + 2 * t
        dp0 = plsc.load_gather(gdp, [k0])
        off0 = plsc.load_gather(goff, [k0])
        dp1 = plsc.load_gather(gdp, [k0 + 1])
        off1 = plsc.load_gather(goff, [k0 + 1])
        for j in range(NCHUNK):
            cidx = lane_iota + (j * L)
            nb0 = plsc.load_gather(nbuf, [off0 + cidx])
            nb1 = plsc.load_gather(nbuf, [off1 + cidx])
            h0 = jnp.maximum(dp0 - nb0 + hinge_c, 0.0)
            h1 = jnp.maximum(dp1 - nb1 + hinge_c, 0.0)
            if j % 2 == 0:
                a0 = a0 + h0
                a2 = a2 + h1
            else:
                a1 = a1 + h0
                a3 = a3 + h1
        return a0, a1, a2, a3

    zero_f = jnp.zeros((L,), jnp.float32)
    accs = lax.fori_loop(0, (tot + 1) // 2, p_body,
                         (zero_f, zero_f, zero_f, zero_f))
    acc = (accs[0] + accs[1]) + (accs[2] + accs[3])
    acc = acc + jnp.float32(EPS / L) * pairs.astype(jnp.float32)
    stage[...] = acc
    pltpu.sync_copy(stage, out_hbm.at[pl.ds(wid * L, L)])


def _tc_share_kernel(d_ref, ycol_ref, yrow_ref, out_ref):
    step = pl.program_id(0)
    dblk = d_ref[:, :]
    ya_col = ycol_ref[:, :]
    yrow = yrow_ref[:, :]
    col_iota = lax.broadcasted_iota(jnp.int32, (TCB, B), 1)
    a_col = (K_SC + step * TCB
             + lax.broadcasted_iota(jnp.int32, (TCB, 1), 0))
    hinge_c = jnp.float32(MARGIN - EPS)
    same = yrow == ya_col
    posm = same & (col_iota != a_col)
    dpos = jnp.where(posm, dblk + hinge_c, jnp.float32(-BIG))
    dneg = jnp.where(same, jnp.float32(BIG), dblk)
    acc = jnp.zeros((TCB, B), jnp.float32)
    for p in range(B):
        acc = acc + jnp.maximum(dpos[:, p:p + 1] - dneg, 0.0)
    npos = jnp.sum(posm.astype(jnp.float32), axis=1, keepdims=True)
    s = jnp.sum(acc) + jnp.float32(EPS) * jnp.sum(npos * (255.0 - npos))
    prev = jnp.where(step == 0, 0.0, out_ref[0, 0])
    out_ref[0, 0] = prev + s


_tc_share = pl.pallas_call(
    _tc_share_kernel,
    grid=(TC_STEPS,),
    in_specs=[
        pl.BlockSpec((TCB, B), lambda s: (K_SC // TCB + s, 0)),
        pl.BlockSpec((TCB, 1), lambda s: (K_SC // TCB + s, 0)),
        pl.BlockSpec((1, B), lambda s: (0, 0)),
    ],
    out_specs=pl.BlockSpec(memory_space=pltpu.SMEM),
    out_shape=jax.ShapeDtypeStruct((1, 1), jnp.float32),
)


def kernel(x, y):
    d = _compute_dists(x)
    sc_partials = _triplet_sc(d, y)
    tc_partial = _tc_share(d, y.reshape(B, 1), y.reshape(1, B))
    return jnp.sum(sc_partials) + tc_partial[0, 0]


# speedup vs baseline: 1.8645x; 1.2510x over previous
import functools

import jax
import jax.numpy as jnp
from jax import lax
from jax.experimental import pallas as pl
from jax.experimental.pallas import tpu as pltpu
from jax.experimental.pallas import tpu_sc as plsc

B = 256
MARGIN = 0.2
EPS = 1e-8
BIG = 1e30

NC = 2
NS = 16
NW = NC * NS
L = 16
NCHUNK = B // L

K_SC = 64
APW = K_SC // NW
PBUF = APW * (B - 1) + 2 * L
TCB = B - K_SC


def _dist_kernel(x_ref, d_ref):
    x = x_ref[:, :]
    g = lax.dot_general(x, x, (((1,), (1,)), ((), ())),
                        preferred_element_type=jnp.float32)
    r = jnp.sum(x * x, axis=1)
    sq = r[:, None] + r[None, :] - 2.0 * g
    d_ref[:, :] = jnp.sqrt(jnp.maximum(sq, 1e-12))


_compute_dists = pl.pallas_call(
    _dist_kernel,
    out_shape=jax.ShapeDtypeStruct((B, B), jnp.float32),
)


@functools.partial(
    pl.kernel,
    out_type=jax.ShapeDtypeStruct((NW * L,), jnp.float32),
    mesh=plsc.VectorSubcoreMesh(core_axis_name="c", subcore_axis_name="s"),
    scratch_types=[
        pltpu.VMEM((APW, B), jnp.float32),
        pltpu.VMEM((B,), jnp.int32),
        pltpu.VMEM((APW * B,), jnp.float32),
        pltpu.VMEM((PBUF,), jnp.float32),
        pltpu.VMEM((PBUF,), jnp.int32),
        pltpu.VMEM((L,), jnp.float32),
    ],
    compiler_params=pltpu.CompilerParams(needs_layout_passes=False),
)
def _triplet_sc(d_hbm, y_hbm, out_hbm, d_v, y_v, nbuf, gdp, goff, stage):
    wid = lax.axis_index("s") * NC + lax.axis_index("c")
    base = wid * APW
    pltpu.sync_copy(y_hbm, y_v)
    pltpu.sync_copy(d_hbm.at[pl.ds(base, APW)], d_v)

    lane_iota = lax.iota(jnp.int32, L)
    zero_i = jnp.zeros((L,), jnp.int32)
    hinge_c = jnp.float32(MARGIN - EPS)

    base_splat = zero_i + base
    ya = [plsc.load_gather(y_v, [base_splat + i]) for i in range(APW)]

    pbases = [zero_i] * APW
    gbase = zero_i
    for j in range(NCHUNK):
        yj = y_v[pl.ds(j * L, L)]
        idxj = lane_iota + (j * L)
        for i in range(APW):
            dj = d_v[i, pl.ds(j * L, L)]
            same = yj == ya[i]
            posm = same & (idxj != base_splat + i)
            nbuf[pl.ds(i * B + j * L, L)] = jnp.where(same, jnp.float32(BIG), dj)
            dest = gbase + plsc.cumsum(posm.astype(jnp.int32)) - 1
            dest = jnp.where(posm, dest, 0)
            plsc.store_scatter(gdp, [dest], dj, mask=posm)
            plsc.store_scatter(goff, [dest], zero_i + (i * B), mask=posm)
            pc = plsc.all_reduce_population_count(posm)
            pbases[i] = pbases[i] + pc
            gbase = gbase + pc

    pairs = zero_i
    for i in range(APW):
        pairs = pairs + pbases[i] * (255 - pbases[i])

    tot = jnp.max(gbase)
    plsc.store_scatter(gdp, [zero_i + tot + lane_iota], jnp.full((L,), -BIG, jnp.float32))
    plsc.store_scatter(goff, [zero_i + tot + lane_iota], zero_i)

    def p_body(t, accs):
        a0, a1, a2, a3 = accs
        k0 = zero_i + 2 * t
        dp0 = plsc.load_gather(gdp, [k0])
        off0 = plsc.load_gather(goff, [k0])
        dp1 = plsc.load_gather(gdp, [k0 + 1])
        off1 = plsc.load_gather(goff, [k0 + 1])
        for j in range(NCHUNK):
            cidx = lane_iota + (j * L)
            nb0 = plsc.load_gather(nbuf, [off0 + cidx])
            nb1 = plsc.load_gather(nbuf, [off1 + cidx])
            h0 = jnp.maximum(dp0 - nb0 + hinge_c, 0.0)
            h1 = jnp.maximum(dp1 - nb1 + hinge_c, 0.0)
            if j % 2 == 0:
                a0 = a0 + h0
                a2 = a2 + h1
            else:
                a1 = a1 + h0
                a3 = a3 + h1
        return a0, a1, a2, a3

    zero_f = jnp.zeros((L,), jnp.float32)
    accs = lax.fori_loop(0, (tot + 1) // 2, p_body,
                         (zero_f, zero_f, zero_f, zero_f))
    acc = (accs[0] + accs[1]) + (accs[2] + accs[3])
    acc = acc + jnp.float32(EPS / L) * pairs.astype(jnp.float32)
    stage[...] = acc
    pltpu.sync_copy(stage, out_hbm.at[pl.ds(wid * L, L)])


def _tc_share_kernel(d_ref, ycol_ref, yrow_ref, out_ref):
    dblk = d_ref[:, :]
    ycol = ycol_ref[:, :]
    ya_row = yrow_ref[:, :]
    b_iota = lax.broadcasted_iota(jnp.int32, (B, B), 0)
    a_row = lax.broadcasted_iota(jnp.int32, (1, B), 1)
    hinge_c = jnp.float32(MARGIN - EPS)
    same = ycol == ya_row
    posm = same & (b_iota != a_row) & (a_row >= K_SC)
    dpos = jnp.where(posm, dblk + hinge_c, jnp.float32(-BIG))
    dneg = jnp.where(same, jnp.float32(BIG), dblk)
    acc = jnp.zeros((B, B), jnp.float32)
    for n in range(B):
        acc = acc + jnp.maximum(dpos - dneg[n:n + 1, :], 0.0)
    npos = jnp.sum(posm.astype(jnp.float32), axis=0, keepdims=True)
    s = jnp.sum(acc) + jnp.float32(EPS) * jnp.sum(npos * (255.0 - npos))
    out_ref[0, 0] = s


_tc_share = pl.pallas_call(
    _tc_share_kernel,
    out_specs=pl.BlockSpec(memory_space=pltpu.SMEM),
    out_shape=jax.ShapeDtypeStruct((1, 1), jnp.float32),
)


def kernel(x, y):
    d = _compute_dists(x)
    sc_partials = _triplet_sc(d, y)
    tc_partial = _tc_share(d, y.reshape(B, 1), y.reshape(1, B))
    return jnp.sum(sc_partials) + tc_partial[0, 0]


# speedup vs baseline: 2.2378x; 1.2002x over previous
import functools

import jax
import jax.numpy as jnp
from jax import lax
from jax.experimental import pallas as pl
from jax.experimental.pallas import tpu as pltpu
from jax.experimental.pallas import tpu_sc as plsc

B = 256
MARGIN = 0.2
EPS = 1e-8
BIG = 1e30

NC = 2
NS = 16
NW = NC * NS
L = 16
NCHUNK = B // L

K_SC = 128
APW = K_SC // NW
PBUF = APW * (B - 1) + 2 * L
TCB = B - K_SC


def _dist_kernel(x_ref, d_ref):
    x = x_ref[:, :]
    g = lax.dot_general(x, x, (((1,), (1,)), ((), ())),
                        preferred_element_type=jnp.float32)
    r = jnp.sum(x * x, axis=1)
    sq = r[:, None] + r[None, :] - 2.0 * g
    d_ref[:, :] = jnp.sqrt(jnp.maximum(sq, 1e-12))


_compute_dists = pl.pallas_call(
    _dist_kernel,
    out_shape=jax.ShapeDtypeStruct((B, B), jnp.float32),
)


@functools.partial(
    pl.kernel,
    out_type=jax.ShapeDtypeStruct((NW * L,), jnp.float32),
    mesh=plsc.VectorSubcoreMesh(core_axis_name="c", subcore_axis_name="s"),
    scratch_types=[
        pltpu.VMEM((APW, B), jnp.float32),
        pltpu.VMEM((B,), jnp.int32),
        pltpu.VMEM((APW * B,), jnp.float32),
        pltpu.VMEM((PBUF,), jnp.float32),
        pltpu.VMEM((PBUF,), jnp.int32),
        pltpu.VMEM((L,), jnp.float32),
    ],
    compiler_params=pltpu.CompilerParams(needs_layout_passes=False),
)
def _triplet_sc(d_hbm, y_hbm, out_hbm, d_v, y_v, nbuf, gdp, goff, stage):
    wid = lax.axis_index("s") * NC + lax.axis_index("c")
    base = wid * APW
    pltpu.sync_copy(y_hbm, y_v)
    pltpu.sync_copy(d_hbm.at[pl.ds(base, APW)], d_v)

    lane_iota = lax.iota(jnp.int32, L)
    zero_i = jnp.zeros((L,), jnp.int32)
    hinge_c = jnp.float32(MARGIN - EPS)

    base_splat = zero_i + base
    ya = [plsc.load_gather(y_v, [base_splat + i]) for i in range(APW)]

    pbases = [zero_i] * APW
    gbase = zero_i
    for j in range(NCHUNK):
        yj = y_v[pl.ds(j * L, L)]
        idxj = lane_iota + (j * L)
        for i in range(APW):
            dj = d_v[i, pl.ds(j * L, L)]
            same = yj == ya[i]
            posm = same & (idxj != base_splat + i)
            nbuf[pl.ds(i * B + j * L, L)] = jnp.where(same, jnp.float32(BIG), dj)
            dest = gbase + plsc.cumsum(posm.astype(jnp.int32)) - 1
            dest = jnp.where(posm, dest, 0)
            plsc.store_scatter(gdp, [dest], dj, mask=posm)
            plsc.store_scatter(goff, [dest], zero_i + (i * B), mask=posm)
            pc = plsc.all_reduce_population_count(posm)
            pbases[i] = pbases[i] + pc
            gbase = gbase + pc

    pairs = zero_i
    for i in range(APW):
        pairs = pairs + pbases[i] * (255 - pbases[i])

    tot = jnp.max(gbase)
    plsc.store_scatter(gdp, [zero_i + tot + lane_iota], jnp.full((L,), -BIG, jnp.float32))
    plsc.store_scatter(goff, [zero_i + tot + lane_iota], zero_i)

    def p_body(t, accs):
        a0, a1, a2, a3 = accs
        k0 = zero_i + 2 * t
        dp0 = plsc.load_gather(gdp, [k0])
        off0 = plsc.load_gather(goff, [k0])
        dp1 = plsc.load_gather(gdp, [k0 + 1])
        off1 = plsc.load_gather(goff, [k0 + 1])
        for j in range(NCHUNK):
            cidx = lane_iota + (j * L)
            nb0 = plsc.load_gather(nbuf, [off0 + cidx])
            nb1 = plsc.load_gather(nbuf, [off1 + cidx])
            h0 = jnp.maximum(dp0 - nb0 + hinge_c, 0.0)
            h1 = jnp.maximum(dp1 - nb1 + hinge_c, 0.0)
            if j % 2 == 0:
                a0 = a0 + h0
                a2 = a2 + h1
            else:
                a1 = a1 + h0
                a3 = a3 + h1
        return a0, a1, a2, a3

    zero_f = jnp.zeros((L,), jnp.float32)
    accs = lax.fori_loop(0, (tot + 1) // 2, p_body,
                         (zero_f, zero_f, zero_f, zero_f))
    acc = (accs[0] + accs[1]) + (accs[2] + accs[3])
    acc = acc + jnp.float32(EPS / L) * pairs.astype(jnp.float32)
    stage[...] = acc
    pltpu.sync_copy(stage, out_hbm.at[pl.ds(wid * L, L)])


def _tc_share_kernel(d_ref, ycol_ref, yrow_ref, out_ref):
    dblk = d_ref[:, :]
    ycol = ycol_ref[:, :]
    ya_row = yrow_ref[:, :]
    b_iota = lax.broadcasted_iota(jnp.int32, (B, TCB), 0)
    a_row = K_SC + lax.broadcasted_iota(jnp.int32, (1, TCB), 1)
    hinge_c = jnp.float32(MARGIN - EPS)
    same = ycol == ya_row
    posm = same & (b_iota != a_row)
    dpos = jnp.where(posm, dblk + hinge_c, jnp.float32(-BIG))
    dneg = jnp.where(same, jnp.float32(BIG), dblk)
    acc = jnp.zeros((B, TCB), jnp.float32)
    for n in range(B):
        acc = acc + jnp.maximum(dpos - dneg[n:n + 1, :], 0.0)
    npos = jnp.sum(posm.astype(jnp.float32), axis=0, keepdims=True)
    s = jnp.sum(acc) + jnp.float32(EPS) * jnp.sum(npos * (255.0 - npos))
    out_ref[0, 0] = s


_tc_share = pl.pallas_call(
    _tc_share_kernel,
    grid=(1,),
    in_specs=[
        pl.BlockSpec((B, TCB), lambda i: (0, 1)),
        pl.BlockSpec((B, 1), lambda i: (0, 0)),
        pl.BlockSpec((1, TCB), lambda i: (0, 1)),
    ],
    out_specs=pl.BlockSpec(memory_space=pltpu.SMEM),
    out_shape=jax.ShapeDtypeStruct((1, 1), jnp.float32),
)


def kernel(x, y):
    d = _compute_dists(x)
    sc_partials = _triplet_sc(d, y)
    tc_partial = _tc_share(d, y.reshape(B, 1), y.reshape(1, B))
    return jnp.sum(sc_partials) + tc_partial[0, 0]
